# baseline (speedup 1.0000x reference)
import jax
import jax.numpy as jnp
from jax import lax
from jax.experimental import pallas as pl
from jax.experimental.pallas import tpu as pltpu
from jax.experimental.pallas import tpu_sc as plsc

_PARENT = (1, 20, 20, 2, 20, 4, 5, 6, 20, 8, 9, 10, 0, 12, 13, 14, 0, 16,
           17, 18, 20, 22, 7, 24, 11)

_B, _C, _V, _T = 512, 3, 25, 300
_TB = 8
_NTB = 38
_NBB = _B // 128
_UNITS = _C * _NTB * _NBB
_NW = 32
_Q, _R = divmod(_UNITS, _NW)
_NB = 4


def _compute(buf):
    @plsc.parallel_loop(0, _TB, 1)
    def do_row(tr):
        for k in range(_TB):
            off = k * 16
            regs = [buf[j, tr, pl.ds(off, 16)] for j in range(_V)]
            for j in range(_V):
                buf[j, tr, pl.ds(off, 16)] = regs[j] - regs[_PARENT[j]]


def _sc_body(x_hbm, out_hbm,
             b0, b1, b2, b3, si0, si1, si2, si3, so0, so1, so2, so3):
    wid = lax.axis_index("s") * 2 + lax.axis_index("c")
    base = wid * _Q + jnp.minimum(wid, _R)
    cnt = _Q + (wid < _R).astype(jnp.int32)
    bufs = (b0, b1, b2, b3)
    sins, souts = (si0, si1, si2, si3), (so0, so1, so2, so3)

    def ref_at(hbm, i):
        u = base + i
        c = u // (_NTB * _NBB)
        r = u % (_NTB * _NBB)
        t0 = pl.multiple_of((r // _NBB) * _TB, _TB)
        bb = pl.multiple_of((r % _NBB) * 128, 128)
        return hbm.at[c, :, pl.ds(t0, _TB), pl.ds(bb, 128)]

    pltpu.make_async_copy(ref_at(x_hbm, 0), bufs[0], sins[0]).start()
    pltpu.make_async_copy(ref_at(x_hbm, 1), bufs[1], sins[1]).start()

    def do_quad(gp, carry):
        for b in range(_NB):
            i = gp * _NB + b
            s2 = (b + 2) % _NB
            @pl.when(jnp.logical_and(i >= 2, i + 2 < cnt))
            def _():
                pltpu.make_async_copy(
                    bufs[s2], ref_at(out_hbm, i - 2), souts[s2]).wait()

            @pl.when(i + 2 < cnt)
            def _():
                pltpu.make_async_copy(
                    ref_at(x_hbm, i + 2), bufs[s2], sins[s2]).start()

            @pl.when(i < cnt)
            def _():
                pltpu.make_async_copy(ref_at(x_hbm, i), bufs[b], sins[b]).wait()
                _compute(bufs[b])
                pltpu.make_async_copy(
                    bufs[b], ref_at(out_hbm, i), souts[b]).start()
        return carry

    lax.fori_loop(0, (_Q + 1 + _NB - 1) // _NB, do_quad, 0)

    for b in range(_NB):
        pltpu.make_async_copy(
            bufs[b], ref_at(out_hbm, cnt - _NB + b), souts[b]).wait()


def kernel(joint_data):
    x = jnp.transpose(joint_data, (1, 2, 3, 0))
    mesh = plsc.VectorSubcoreMesh(core_axis_name="c", subcore_axis_name="s")
    f = pl.kernel(
        _sc_body,
        mesh=mesh,
        out_type=jax.ShapeDtypeStruct((_C, _V, _T, _B), jnp.float32),
        scratch_types=(
            [pltpu.VMEM((_V, _TB, 128), jnp.float32) for _ in range(_NB)]
            + [pltpu.SemaphoreType.DMA for _ in range(2 * _NB)]
        ),
    )
    out = f(x)
    return jnp.transpose(out, (3, 0, 1, 2))


# speedup vs baseline: 1.0394x; 1.0394x over previous
import jax
import jax.numpy as jnp
from jax import lax
from jax.experimental import pallas as pl
from jax.experimental.pallas import tpu as pltpu
from jax.experimental.pallas import tpu_sc as plsc

_PARENT = (1, 20, 20, 2, 20, 4, 5, 6, 20, 8, 9, 10, 0, 12, 13, 14, 0, 16,
           17, 18, 20, 22, 7, 24, 11)

_B, _C, _V, _T = 512, 3, 25, 300
_UNITS = _C * _T
_NW = 32
_Q, _R = divmod(_UNITS, _NW)
_NI = 4
_NO = 3
_STEP = 12

_CHUNKS = _B // 16


def _compute(xbuf, obuf):
    @plsc.parallel_loop(0, _CHUNKS, 1)
    def do_chunk(k):
        off = pl.multiple_of(k * 16, 16)
        regs = [xbuf[j, pl.ds(off, 16)] for j in range(_V)]
        for j in range(_V):
            obuf[j, pl.ds(off, 16)] = regs[j] - regs[_PARENT[j]]


def _sc_body(x_hbm, out_hbm,
             xb0, xb1, xb2, xb3, ob0, ob1, ob2,
             si0, si1, si2, si3, so0, so1, so2):
    wid = lax.axis_index("s") * 2 + lax.axis_index("c")
    base = wid * _Q + jnp.minimum(wid, _R)
    cnt = _Q + (wid < _R).astype(jnp.int32)
    xbufs, obufs = (xb0, xb1, xb2, xb3), (ob0, ob1, ob2)
    sins, souts = (si0, si1, si2, si3), (so0, so1, so2)

    def src(i):
        u = base + i
        return x_hbm.at[u // _T, :, u % _T]

    def dst(i):
        u = base + i
        return out_hbm.at[u // _T, :, u % _T]

    for p in range(_NI - 1):
        pltpu.make_async_copy(src(p), xbufs[p], sins[p]).start()

    def do_block(gp, carry):
        for b in range(_STEP):
            i = gp * _STEP + b
            xs, os = b % _NI, b % _NO
            @pl.when(i + _NI - 1 < cnt)
            def _():
                pltpu.make_async_copy(
                    src(i + _NI - 1), xbufs[(xs + _NI - 1) % _NI],
                    sins[(xs + _NI - 1) % _NI]).start()

            @pl.when(i < cnt)
            def _():
                pltpu.make_async_copy(src(i), xbufs[xs], sins[xs]).wait()

            @pl.when(jnp.logical_and(i >= _NO, i < cnt))
            def _():
                pltpu.make_async_copy(obufs[os], dst(i - _NO), souts[os]).wait()

            @pl.when(i < cnt)
            def _():
                _compute(xbufs[xs], obufs[os])
                pltpu.make_async_copy(obufs[os], dst(i), souts[os]).start()
        return carry

    lax.fori_loop(0, (_Q + 1 + _STEP - 1) // _STEP, do_block, 0)

    for b in range(_NO):
        pltpu.make_async_copy(obufs[b], dst(b), souts[b]).wait()


def kernel(joint_data):
    x = jnp.transpose(joint_data, (1, 2, 3, 0))
    mesh = plsc.VectorSubcoreMesh(core_axis_name="c", subcore_axis_name="s")
    f = pl.kernel(
        _sc_body,
        mesh=mesh,
        out_type=jax.ShapeDtypeStruct((_C, _V, _T, _B), jnp.float32),
        scratch_types=(
            [pltpu.VMEM((_V, _B), jnp.float32) for _ in range(_NI + _NO)]
            + [pltpu.SemaphoreType.DMA for _ in range(_NI + _NO)]
        ),
    )
    out = f(x)
    return jnp.transpose(out, (3, 0, 1, 2))


# speedup vs baseline: 1.1229x; 1.0803x over previous
import jax
import jax.numpy as jnp
from jax import lax
from jax.experimental import pallas as pl
from jax.experimental.pallas import tpu as pltpu
from jax.experimental.pallas import tpu_sc as plsc

_PARENT = (1, 20, 20, 2, 20, 4, 5, 6, 20, 8, 9, 10, 0, 12, 13, 14, 0, 16,
           17, 18, 20, 22, 7, 24, 11)

_B, _C, _V, _T = 512, 3, 25, 300
_UNITS = _C * _T
_NW = 32
_Q, _R = divmod(_UNITS, _NW)
_NB = 3

_CHUNKS = _B // 16


def _compute(xbuf, obuf):
    @plsc.parallel_loop(0, _CHUNKS, 1)
    def do_chunk(k):
        off = pl.multiple_of(k * 16, 16)
        regs = [xbuf[j, pl.ds(off, 16)] for j in range(_V)]
        for j in range(_V):
            obuf[j, pl.ds(off, 16)] = regs[j] - regs[_PARENT[j]]


def _sc_body(x_hbm, out_hbm,
             xb0, xb1, xb2, ob0, ob1, ob2,
             si0, si1, si2, so0, so1, so2):
    wid = lax.axis_index("s") * 2 + lax.axis_index("c")
    base = wid * _Q + jnp.minimum(wid, _R)
    cnt = _Q + (wid < _R).astype(jnp.int32)
    xbufs, obufs = (xb0, xb1, xb2), (ob0, ob1, ob2)
    sins, souts = (si0, si1, si2), (so0, so1, so2)

    def src(i):
        u = base + i
        return x_hbm.at[u // _T, :, u % _T]

    def dst(i):
        u = base + i
        return out_hbm.at[u // _T, :, u % _T]

    pltpu.make_async_copy(src(0), xbufs[0], sins[0]).start()
    pltpu.make_async_copy(src(1), xbufs[1], sins[1]).start()

    def do_triple(gp, carry):
        for b in range(_NB):
            i = gp * _NB + b
            @pl.when(i + 2 < cnt)
            def _():
                pltpu.make_async_copy(
                    src(i + 2), xbufs[(b + 2) % _NB], sins[(b + 2) % _NB]
                ).start()

            @pl.when(i < cnt)
            def _():
                pltpu.make_async_copy(src(i), xbufs[b], sins[b]).wait()

            @pl.when(jnp.logical_and(i >= _NB, i < cnt))
            def _():
                pltpu.make_async_copy(obufs[b], dst(i - _NB), souts[b]).wait()

            @pl.when(i < cnt)
            def _():
                _compute(xbufs[b], obufs[b])
                pltpu.make_async_copy(obufs[b], dst(i), souts[b]).start()
        return carry

    lax.fori_loop(0, (_Q + 1 + _NB - 1) // _NB + 1, do_triple, 0)

    for b in range(_NB):
        pltpu.make_async_copy(obufs[b], dst(cnt - _NB + b), souts[b]).wait()


def kernel(joint_data):
    x = jnp.transpose(joint_data, (1, 2, 3, 0))
    mesh = plsc.VectorSubcoreMesh(core_axis_name="c", subcore_axis_name="s")
    f = pl.kernel(
        _sc_body,
        mesh=mesh,
        out_type=jax.ShapeDtypeStruct((_C, _V, _T, _B), jnp.float32),
        scratch_types=(
            [pltpu.VMEM((_V, _B), jnp.float32) for _ in range(2 * _NB)]
            + [pltpu.SemaphoreType.DMA for _ in range(2 * _NB)]
        ),
    )
    out = f(x)
    return jnp.transpose(out, (3, 0, 1, 2))
